# Initial kernel scaffold; baseline (speedup 1.0000x reference)
import jax
import jax.numpy as jnp
from jax import lax
from jax.experimental import pallas as pl
from jax.experimental.pallas import tpu as pltpu
from jax.experimental.pallas import tpu_sc as plsc

N, P, C = 10000, 32, 128
NW = 32
NPAD = 10240
VPW = NPAD // NW
VB = 8
CHUNKS = VPW // VB
EPC = VB * P
IDX_B = 128
NIDX = EPC // IDX_B
L = 16
CV = C // L


def _sc_kernel(idx_hbm, w_hbm, table_hbm, out_hbm, idx_v, w_v, rows_v, out_v, sem):
    wid = lax.axis_index("s") * 2 + lax.axis_index("c")
    vbase0 = wid * VPW
    pltpu.sync_copy(idx_hbm.at[wid], idx_v)

    def chunk_body(g, carry):
        vbase = vbase0 + g * VB
        ebase = vbase * P
        pltpu.sync_copy(w_hbm.at[pl.ds(ebase * 4, EPC * 4)], w_v)
        for j in range(NIDX):
            pltpu.async_copy(
                table_hbm.at[idx_v.at[g * NIDX + j]],
                rows_v.at[pl.ds(j * IDX_B, IDX_B)],
                sem,
            ).wait()

        def v_body(v, carry2):
            accs = [jnp.zeros((L,), jnp.float32) for _ in range(4 * CV)]
            row_base = v * P
            woff_base = v * (P * 4)
            for p in range(P):
                ws = [
                    plsc.load_gather(
                        w_v,
                        [jnp.full((L,), woff_base + p * 4 + k, jnp.int32)],
                    )
                    for k in range(4)
                ]
                for c8 in range(CV):
                    row = rows_v[row_base + p, pl.ds(c8 * L, L)]
                    for k in range(4):
                        accs[k * CV + c8] = accs[k * CV + c8] + ws[k] * row
            for k in range(4):
                for c8 in range(CV):
                    out_v[v * 4 + k, pl.ds(c8 * L, L)] = accs[k * CV + c8]
            return carry2

        lax.fori_loop(0, VB, v_body, 0)
        pltpu.sync_copy(out_v, out_hbm.at[pl.ds(vbase * 4, VB * 4)])
        return carry

    lax.fori_loop(0, CHUNKS, chunk_body, 0)


@jax.jit
def kernel(signal_0, kernel_0, kernel_1, patches_idx):
    table = signal_0[0, :, 0, :]
    w = jnp.concatenate(
        [kernel_0[0, :, :, :, 0], kernel_1[0, :, :, :, 0]], axis=-1
    )
    idx = patches_idx[0, :, :, 1]

    pad_v = NPAD - N
    w = jnp.pad(w, ((0, pad_v), (0, 0), (0, 0)))
    idx = jnp.pad(idx, ((0, pad_v), (0, 0)))

    idx_r = idx.reshape(NW, CHUNKS * NIDX, IDX_B)
    w_flat = w.reshape(NPAD * P * 4)

    mesh = plsc.VectorSubcoreMesh(core_axis_name="c", subcore_axis_name="s")
    out = pl.kernel(
        _sc_kernel,
        out_type=jax.ShapeDtypeStruct((NPAD * 4, C), jnp.float32),
        mesh=mesh,
        scratch_types=[
            pltpu.VMEM((CHUNKS * NIDX, IDX_B), jnp.int32),
            pltpu.VMEM((EPC * 4,), jnp.float32),
            pltpu.VMEM((EPC, C), jnp.float32),
            pltpu.VMEM((VB * 4, C), jnp.float32),
            pltpu.SemaphoreType.DMA,
        ],
    )(idx_r, w_flat, table)

    y = out.reshape(NPAD, 4, C)[:N]
    y0 = y[None, :, 0:1, :]
    y1 = y[None, :, 1:4, :]
    return (y0, y1)


# speedup vs baseline: 2.6667x; 2.6667x over previous
import jax
import jax.numpy as jnp
from jax import lax
from jax.experimental import pallas as pl
from jax.experimental.pallas import tpu as pltpu
from jax.experimental.pallas import tpu_sc as plsc

N, P, C = 10000, 32, 128
NW = 32
NPAD = 10240
VPW = NPAD // NW
VB = 8
CHUNKS = VPW // VB
EPC = VB * P
IDX_B = 128
NIDX = EPC // IDX_B
L = 16
CV = C // L


_SPLAT_DNUMS = lax.GatherDimensionNumbers(
    offset_dims=(), collapsed_slice_dims=(0,), start_index_map=(0,)
)


def _splat(vec, elt):
    return lax.gather(
        vec,
        jnp.full((L, 1), elt, jnp.int32),
        _SPLAT_DNUMS,
        slice_sizes=(1,),
        mode=lax.GatherScatterMode.PROMISE_IN_BOUNDS,
    )


def _sc_kernel(idx_hbm, w_hbm, table_hbm, out_hbm, idx_v, w_v, rows_v, out_v, sem):
    wid = lax.axis_index("s") * 2 + lax.axis_index("c")
    vbase0 = wid * VPW
    pltpu.sync_copy(idx_hbm.at[wid], idx_v)

    def chunk_body(g, carry):
        vbase = vbase0 + g * VB
        ebase = vbase * P
        pltpu.sync_copy(w_hbm.at[pl.ds(ebase * 4, EPC * 4)], w_v)
        for j in range(NIDX):
            pltpu.async_copy(
                table_hbm.at[idx_v.at[g * NIDX + j]],
                rows_v.at[pl.ds(j * IDX_B, IDX_B)],
                sem,
            ).wait()

        def v_body(v, carry2):
            accs = [jnp.zeros((L,), jnp.float32) for _ in range(4 * CV)]
            row_base = v * P
            woff_base = v * (P * 4)
            wvecs = [w_v[pl.ds(woff_base + j * L, L)] for j in range(CV)]
            for p in range(P):
                wv = wvecs[p // 4]
                ws = [_splat(wv, (p % 4) * 4 + k) for k in range(4)]
                for c8 in range(CV):
                    row = rows_v[row_base + p, pl.ds(c8 * L, L)]
                    for k in range(4):
                        accs[k * CV + c8] = accs[k * CV + c8] + ws[k] * row
            for k in range(4):
                for c8 in range(CV):
                    out_v[v * 4 + k, pl.ds(c8 * L, L)] = accs[k * CV + c8]
            return carry2

        lax.fori_loop(0, VB, v_body, 0)
        pltpu.sync_copy(out_v, out_hbm.at[pl.ds(vbase * 4, VB * 4)])
        return carry

    lax.fori_loop(0, CHUNKS, chunk_body, 0)


@jax.jit
def kernel(signal_0, kernel_0, kernel_1, patches_idx):
    table = signal_0[0, :, 0, :]
    w = jnp.concatenate(
        [kernel_0[0, :, :, :, 0], kernel_1[0, :, :, :, 0]], axis=-1
    )
    idx = patches_idx[0, :, :, 1]

    pad_v = NPAD - N
    w = jnp.pad(w, ((0, pad_v), (0, 0), (0, 0)))
    idx = jnp.pad(idx, ((0, pad_v), (0, 0)))

    idx_r = idx.reshape(NW, CHUNKS * NIDX, IDX_B)
    w_flat = w.reshape(NPAD * P * 4)

    mesh = plsc.VectorSubcoreMesh(core_axis_name="c", subcore_axis_name="s")
    out = pl.kernel(
        _sc_kernel,
        out_type=jax.ShapeDtypeStruct((NPAD * 4, C), jnp.float32),
        mesh=mesh,
        scratch_types=[
            pltpu.VMEM((CHUNKS * NIDX, IDX_B), jnp.int32),
            pltpu.VMEM((EPC * 4,), jnp.float32),
            pltpu.VMEM((EPC, C), jnp.float32),
            pltpu.VMEM((VB * 4, C), jnp.float32),
            pltpu.SemaphoreType.DMA,
        ],
    )(idx_r, w_flat, table)

    y = out.reshape(NPAD, 4, C)[:N]
    y0 = y[None, :, 0:1, :]
    y1 = y[None, :, 1:4, :]
    return (y0, y1)


# speedup vs baseline: 6.7280x; 2.5230x over previous
import jax
import jax.numpy as jnp
from jax import lax
from jax.experimental import pallas as pl
from jax.experimental.pallas import tpu as pltpu
from jax.experimental.pallas import tpu_sc as plsc

N, P, C = 10000, 32, 128
NW = 32
NPAD = 10240
VPW = NPAD // NW
VB = 8
CHUNKS = VPW // VB
EPC = VB * P
IDX_B = 128
NIDX = EPC // IDX_B
L = 16
CV = C // L

_SPLAT_DNUMS = lax.GatherDimensionNumbers(
    offset_dims=(), collapsed_slice_dims=(0,), start_index_map=(0,)
)


def _splat(vec, elt):
    return lax.gather(
        vec,
        jnp.full((L, 1), elt, jnp.int32),
        _SPLAT_DNUMS,
        slice_sizes=(1,),
        mode=lax.GatherScatterMode.PROMISE_IN_BOUNDS,
    )


def _sc_kernel(idx_hbm, w0_hbm, w1_hbm, table_hbm, out_hbm,
               idx_v, w0_v, w1_v, rows_v, out_v, sem0, sem1):
    wid = lax.axis_index("s") * 2 + lax.axis_index("c")
    vbase0 = pl.multiple_of(jnp.minimum(wid * VPW, N - VPW), 16)
    pltpu.sync_copy(idx_hbm.at[pl.ds(pl.multiple_of(vbase0 * P, 256), VPW * P)],
                    idx_v)
    sems = (sem0, sem1)

    def issue(g, b):
        ebase = pl.multiple_of((vbase0 + g * VB) * P, 256)
        sem = sems[b]
        for j in range(NIDX):
            pltpu.async_copy(
                table_hbm.at[idx_v.at[pl.ds((g * NIDX + j) * IDX_B, IDX_B)]],
                rows_v.at[pl.ds((b * NIDX + j) * IDX_B, IDX_B)],
                sem,
            )
        pltpu.async_copy(w0_hbm.at[pl.ds(ebase, EPC)],
                         w0_v.at[pl.ds(b * EPC, EPC)], sem)
        pltpu.async_copy(w1_hbm.at[pl.ds(ebase * 3, EPC * 3)],
                         w1_v.at[pl.ds(b * EPC * 3, EPC * 3)], sem)

    def wait(g, b):
        ebase = pl.multiple_of((vbase0 + g * VB) * P, 256)
        sem = sems[b]
        for j in range(NIDX):
            pltpu.make_async_copy(
                table_hbm.at[idx_v.at[pl.ds((g * NIDX + j) * IDX_B, IDX_B)]],
                rows_v.at[pl.ds((b * NIDX + j) * IDX_B, IDX_B)],
                sem,
            ).wait()
        pltpu.make_async_copy(w0_hbm.at[pl.ds(ebase, EPC)],
                              w0_v.at[pl.ds(b * EPC, EPC)], sem).wait()
        pltpu.make_async_copy(w1_hbm.at[pl.ds(ebase * 3, EPC * 3)],
                              w1_v.at[pl.ds(b * EPC * 3, EPC * 3)], sem).wait()

    def compute(g, b):
        def v_body(v, carry):
            accs = [jnp.zeros((L,), jnp.float32) for _ in range(4 * CV)]
            row_base = b * EPC + v * P
            w0base = b * EPC + v * P
            w1base = (b * EPC + v * P) * 3
            w0vecs = [w0_v[pl.ds(w0base + j * L, L)] for j in range(P // L)]
            w1vecs = [w1_v[pl.ds(w1base + j * L, L)] for j in range(P * 3 // L)]
            for p in range(P):
                ws = [_splat(w0vecs[p // L], p % L)]
                for k in range(3):
                    off = p * 3 + k
                    ws.append(_splat(w1vecs[off // L], off % L))
                for c8 in range(CV):
                    row = rows_v[row_base + p, pl.ds(c8 * L, L)]
                    for k in range(4):
                        accs[k * CV + c8] = accs[k * CV + c8] + ws[k] * row
            for k in range(4):
                for c8 in range(CV):
                    out_v[v * 4 + k, pl.ds(c8 * L, L)] = accs[k * CV + c8]
            return carry

        lax.fori_loop(0, VB, v_body, 0)
        pltpu.sync_copy(
            out_v,
            out_hbm.at[pl.ds(pl.multiple_of((vbase0 + g * VB) * 4, 32), VB * 4)],
        )

    issue(0, 0)

    def loop_body(gg, carry):
        g0 = gg * 2
        wait(g0, 0)
        issue(g0 + 1, 1)
        compute(g0, 0)
        wait(g0 + 1, 1)

        @pl.when(g0 + 2 < CHUNKS)
        def _():
            issue(g0 + 2, 0)

        compute(g0 + 1, 1)
        return carry

    lax.fori_loop(0, CHUNKS // 2, loop_body, 0)


@jax.jit
def kernel(signal_0, kernel_0, kernel_1, patches_idx):
    table = signal_0[0, :, 0, :]
    w0 = kernel_0.reshape(N * P)
    w1 = kernel_1.reshape(N * P * 3)
    idx = patches_idx[0, :, :, 1].reshape(N * P)

    mesh = plsc.VectorSubcoreMesh(core_axis_name="c", subcore_axis_name="s")
    out = pl.kernel(
        _sc_kernel,
        out_type=jax.ShapeDtypeStruct((N * 4, C), jnp.float32),
        mesh=mesh,
        scratch_types=[
            pltpu.VMEM((VPW * P,), jnp.int32),
            pltpu.VMEM((2 * EPC,), jnp.float32),
            pltpu.VMEM((2 * EPC * 3,), jnp.float32),
            pltpu.VMEM((2 * EPC, C), jnp.float32),
            pltpu.VMEM((VB * 4, C), jnp.float32),
            pltpu.SemaphoreType.DMA,
            pltpu.SemaphoreType.DMA,
        ],
    )(idx, w0, w1, table)

    y = out.reshape(N, 4, C)
    y0 = y[None, :, 0:1, :]
    y1 = y[None, :, 1:4, :]
    return (y0, y1)
